# baseline (speedup 1.0000x reference)
import functools

import jax
import jax.numpy as jnp
from jax import lax
from jax.experimental import pallas as pl
from jax.experimental.pallas import tpu as pltpu
from jax.experimental.pallas import tpu_sc as plsc

_EPS = 1e-5
_NC = 2
_NS = 16
_NW = _NC * _NS
_CH = 128


def _worker_mesh():
    return plsc.VectorSubcoreMesh(core_axis_name="c", subcore_axis_name="s",
                                  num_cores=_NC, num_subcores=_NS)


def _sc_gather(table, idx3, ep):
    nw, c, ch = idx3.shape
    d = table.shape[1]
    epw = ep // _NW

    @functools.partial(
        pl.kernel,
        out_type=jax.ShapeDtypeStruct((ep, d), jnp.float32),
        mesh=_worker_mesh(),
        compiler_params=pltpu.CompilerParams(use_tc_tiling_on_sc=False),
        scratch_types=[
            pltpu.VMEM((c, ch), jnp.int32),
            pltpu.VMEM((ch, d), jnp.float32),
            pltpu.VMEM((ch, d), jnp.float32),
            pltpu.SemaphoreType.DMA,
            pltpu.SemaphoreType.DMA,
        ],
    )
    def k(table_hbm, idx_hbm, out_hbm, idx_v, u0, u1, sg0, sg1):
        cid = lax.axis_index("c")
        sid = lax.axis_index("s")
        wid = sid * _NC + cid
        base = wid * epw
        pltpu.sync_copy(idx_hbm.at[wid], idx_v)

        def start(k_, u, sg):
            pltpu.async_copy(table_hbm.at[idx_v.at[k_]], u, sg)

        def finish(k_, u, sg):
            pltpu.make_async_copy(table_hbm.at[idx_v.at[k_]], u, sg).wait()
            pltpu.sync_copy(u, out_hbm.at[pl.ds(base + k_ * ch, ch)])

        start(0, u0, sg0)

        @pl.loop(0, (c - 1) // 2)
        def _(j):
            k0 = 2 * j
            start(k0 + 1, u1, sg1)
            finish(k0, u0, sg0)
            start(k0 + 2, u0, sg0)
            finish(k0 + 1, u1, sg1)

        finish(c - 1, u0, sg0)

    return k(table, idx3)


def _sc_scatter(msg, dst3, zeros, n_acc, ep):
    nw, c, ch = dst3.shape
    epw = ep // _NW
    rpt = n_acc // _NS

    @functools.partial(
        pl.kernel,
        out_type=jax.ShapeDtypeStruct((_NC, n_acc, 16), jnp.float32),
        mesh=_worker_mesh(),
        compiler_params=pltpu.CompilerParams(use_tc_tiling_on_sc=False),
        scratch_types=[
            pltpu.VMEM((c, ch), jnp.int32),
            pltpu.VMEM((ch, 16), jnp.float32),
            pltpu.VMEM((ch, 16), jnp.float32),
            pltpu.VMEM_SHARED((n_acc, 16), jnp.float32),
            pltpu.SemaphoreType.DMA,
            pltpu.SemaphoreType.DMA,
        ],
    )
    def k(msg_hbm, dst_hbm, zero_hbm, out_hbm, idx_v, buf0, buf1, acc_sh,
          sem0, sem1):
        cid = lax.axis_index("c")
        sid = lax.axis_index("s")
        wid = sid * _NC + cid
        base = wid * epw
        pltpu.sync_copy(dst_hbm.at[wid], idx_v)
        pltpu.sync_copy(zero_hbm.at[pl.ds(sid * rpt, rpt)],
                        acc_sh.at[pl.ds(sid * rpt, rpt)])
        plsc.subcore_barrier()

        pltpu.async_copy(msg_hbm.at[pl.ds(base, ch)], buf0, sem0)

        @pl.loop(0, (c - 1) // 2)
        def _(j):
            k0 = 2 * j
            pltpu.async_copy(msg_hbm.at[pl.ds(base + (k0 + 1) * ch, ch)],
                             buf1, sem1)
            pltpu.make_async_copy(msg_hbm.at[pl.ds(base + k0 * ch, ch)],
                                  buf0, sem0).wait()
            pltpu.sync_copy(buf0, acc_sh.at[idx_v.at[k0]], add=True)
            pltpu.async_copy(msg_hbm.at[pl.ds(base + (k0 + 2) * ch, ch)],
                             buf0, sem0)
            pltpu.make_async_copy(msg_hbm.at[pl.ds(base + (k0 + 1) * ch, ch)],
                                  buf1, sem1).wait()
            pltpu.sync_copy(buf1, acc_sh.at[idx_v.at[k0 + 1]], add=True)

        pltpu.make_async_copy(msg_hbm.at[pl.ds(base + (c - 1) * ch, ch)],
                              buf0, sem0).wait()
        pltpu.sync_copy(buf0, acc_sh.at[idx_v.at[c - 1]], add=True)

        plsc.subcore_barrier()
        pltpu.sync_copy(acc_sh.at[pl.ds(sid * rpt, rpt)],
                        out_hbm.at[cid, pl.ds(sid * rpt, rpt)])

    return k(msg, dst3, zeros)


def _edge_msgs(hsrc_p, attr_t, w1, b1, w2, b2, in_dim, blk, ep):
    kdim = w2.shape[1]
    h_out = kdim // in_dim
    slots = 128 // in_dim
    grid = ep // blk
    pr = blk // slots

    ii = jnp.arange(kdim) // h_out
    exps = jnp.zeros((slots, 128, kdim), jnp.float32)
    for p in range(slots):
        exps = exps.at[p, p * in_dim + ii, jnp.arange(kdim)].set(1.0)
    exps = exps.astype(jnp.bfloat16)
    oo = jnp.arange(kdim) % h_out
    select = (oo[:, None] == jnp.arange(h_out)[None, :]).astype(jnp.bfloat16)

    def body(hsrc_ref, attr_ref, exp_ref, w1_ref, b1_ref, w2_ref, b2_ref,
             sel_ref, msg_ref):
        hb = hsrc_ref[...].astype(jnp.bfloat16)
        parts = [jnp.dot(hb, exp_ref[p], preferred_element_type=jnp.float32)
                 for p in range(slots)]
        hexp = jnp.concatenate(parts, axis=0)
        a = attr_ref[...].T.astype(jnp.bfloat16)
        z = jnp.dot(a, w1_ref[...],
                    preferred_element_type=jnp.float32) + b1_ref[...]
        u = jnp.maximum(z, 0.0).astype(jnp.bfloat16)
        wmat = jnp.dot(u, w2_ref[...],
                       preferred_element_type=jnp.float32) + b2_ref[...]
        prod = (hexp * wmat).astype(jnp.bfloat16)
        msg_ref[...] = jnp.dot(prod, sel_ref[...],
                               preferred_element_type=jnp.float32)

    full = lambda shape: pl.BlockSpec(shape, lambda i: (0,) * len(shape))
    return pl.pallas_call(
        body,
        grid=(grid,),
        in_specs=[
            pl.BlockSpec((pr, 128), lambda i: (i, 0)),
            pl.BlockSpec((4, blk), lambda i: (0, i)),
            full(exps.shape),
            full(w1.shape),
            full((1, kdim)),
            full(w2.shape),
            full((1, kdim)),
            full(select.shape),
        ],
        out_specs=pl.BlockSpec((blk, h_out), lambda i: (i, 0)),
        out_shape=jax.ShapeDtypeStruct((ep, h_out), jnp.float32),
    )(hsrc_p, attr_t, exps, w1.astype(jnp.bfloat16), b1.reshape(1, -1),
      w2.astype(jnp.bfloat16), b2.reshape(1, -1), select)


def _bn_layer(aggp, h, root, bias, gamma, beta, n):
    h_dim = root.shape[1]

    def body(agg_ref, h_ref, root_ref, bias_ref, gamma_ref, beta_ref,
             hout_ref, colsum_ref):
        agg = agg_ref[0, :n, :] + agg_ref[1, :n, :]
        hpre = agg + jnp.dot(h_ref[...], root_ref[...],
                             preferred_element_type=jnp.float32) + bias_ref[...]
        mean = jnp.mean(hpre, axis=0, keepdims=True)
        var = jnp.mean((hpre - mean) ** 2, axis=0, keepdims=True)
        hn = (hpre - mean) * lax.rsqrt(var + _EPS) * gamma_ref[...] + beta_ref[...]
        hout = jnp.maximum(hn, 0.0)
        hout_ref[...] = hout
        colsum_ref[...] = jnp.sum(hout, axis=0, keepdims=True)

    return pl.pallas_call(
        body,
        out_shape=(jax.ShapeDtypeStruct((n, h_dim), jnp.float32),
                   jax.ShapeDtypeStruct((1, h_dim), jnp.float32)),
    )(aggp, h, root, bias.reshape(1, -1), gamma.reshape(1, -1),
      beta.reshape(1, -1))


def _head(colsum_cat, jump_w, jump_b, reg_w1, reg_b1, reg_w2, reg_b2, n):
    def body(cs_ref, jw_ref, jb_ref, w1_ref, b1_ref, w2_ref, b2_ref, out_ref):
        pooled = jnp.dot(cs_ref[...], jw_ref[...],
                         preferred_element_type=jnp.float32) + n * jb_ref[...]
        r = jnp.maximum(jnp.dot(pooled, w1_ref[...],
                                preferred_element_type=jnp.float32)
                        + b1_ref[...], 0.0)
        out_ref[...] = jnp.dot(r, w2_ref[...],
                               preferred_element_type=jnp.float32) + b2_ref[...]

    return pl.pallas_call(
        body,
        out_shape=jax.ShapeDtypeStruct((1, 1), jnp.float32),
    )(colsum_cat, jump_w, jump_b.reshape(1, -1), reg_w1, reg_b1.reshape(1, -1),
      reg_w2, reg_b2.reshape(1, -1))


def kernel(x, edge_index, edge_attr, params):
    n = x.shape[0]
    e = edge_attr.shape[0]
    src = edge_index[0]
    dst = edge_index[1]

    ep = -(-e // 4096) * 4096
    epw = ep // _NW
    c = epw // _CH
    n_acc = ((n + 1 + _NS - 1) // _NS) * _NS

    src3 = jnp.pad(src, (0, ep - e)).reshape(_NW, c, _CH)
    dst_pad = jnp.pad(dst, (0, ep - e), constant_values=n)
    zeros_acc = jnp.zeros((n_acc, 16), jnp.float32)
    attr_t = jnp.pad(edge_attr.T, ((0, 0), (0, ep - e)))

    h = x
    colsums = []
    for li, lp in enumerate(params["layers"]):
        in_dim = h.shape[1]
        slots = 128 // in_dim
        blk = 1024 if li == 0 else 4096
        pr = blk // slots
        hsrc = _sc_gather(h, src3, ep)
        hsrc_p = hsrc.reshape(ep * in_dim // 128, 128)
        perm = lambda v: v.reshape(-1, ep // blk, pr, slots) \
                          .swapaxes(2, 3).reshape(v.shape[0], ep) \
            if v.ndim == 2 else \
            v.reshape(ep // blk, pr, slots).swapaxes(1, 2).reshape(ep)
        msg = _edge_msgs(hsrc_p, perm(attr_t), lp["w1"], lp["b1"], lp["w2"],
                         lp["b2"], in_dim, blk, ep)
        dst3 = perm(dst_pad).reshape(_NW, c, _CH)
        aggp = _sc_scatter(msg, dst3, zeros_acc, n_acc, ep)
        h, cs = _bn_layer(aggp, h, lp["root"], lp["bias"], lp["gamma"],
                          lp["beta"], n)
        colsums.append(cs)

    cs_cat = jnp.concatenate(colsums, axis=1)
    return _head(cs_cat, params["jump_w"], params["jump_b"],
                 params["reg_w1"], params["reg_b1"],
                 params["reg_w2"], params["reg_b2"], float(n))


# speedup vs baseline: 1.1337x; 1.1337x over previous
import functools

import jax
import jax.numpy as jnp
import numpy as np
from jax import lax
from jax.experimental import pallas as pl
from jax.experimental.pallas import tpu as pltpu
from jax.experimental.pallas import tpu_sc as plsc

_EPS = 1e-5
_NC = 2
_NS = 16
_NW = _NC * _NS
_CH = 128


def _worker_mesh():
    return plsc.VectorSubcoreMesh(core_axis_name="c", subcore_axis_name="s",
                                  num_cores=_NC, num_subcores=_NS)


def _sc_gather(table, idx3, ep):
    nw, c, ch = idx3.shape
    d = table.shape[1]
    epw = ep // _NW

    @functools.partial(
        pl.kernel,
        out_type=jax.ShapeDtypeStruct((ep, d), jnp.float32),
        mesh=_worker_mesh(),
        compiler_params=pltpu.CompilerParams(use_tc_tiling_on_sc=False),
        scratch_types=[
            pltpu.VMEM((c, ch), jnp.int32),
            pltpu.VMEM((ch, d), jnp.float32),
            pltpu.VMEM((ch, d), jnp.float32),
            pltpu.SemaphoreType.DMA,
            pltpu.SemaphoreType.DMA,
        ],
    )
    def k(table_hbm, idx_hbm, out_hbm, idx_v, u0, u1, sg0, sg1):
        cid = lax.axis_index("c")
        sid = lax.axis_index("s")
        wid = sid * _NC + cid
        base = wid * epw
        pltpu.sync_copy(idx_hbm.at[wid], idx_v)

        def start(k_, u, sg):
            pltpu.async_copy(table_hbm.at[idx_v.at[k_]], u, sg)

        def finish(k_, u, sg):
            pltpu.make_async_copy(table_hbm.at[idx_v.at[k_]], u, sg).wait()
            pltpu.sync_copy(u, out_hbm.at[pl.ds(base + k_ * ch, ch)])

        start(0, u0, sg0)

        @pl.loop(0, (c - 1) // 2)
        def _(j):
            k0 = 2 * j
            start(k0 + 1, u1, sg1)
            finish(k0, u0, sg0)
            start(k0 + 2, u0, sg0)
            finish(k0 + 1, u1, sg1)

        finish(c - 1, u0, sg0)

    return k(table, idx3)


def _sc_scatter(msg, dst3, zeros, n_acc, ep):
    nw, c, ch = dst3.shape
    epw = ep // _NW
    rpt = n_acc // _NS

    @functools.partial(
        pl.kernel,
        out_type=jax.ShapeDtypeStruct((_NC, n_acc, 16), jnp.float32),
        mesh=_worker_mesh(),
        compiler_params=pltpu.CompilerParams(use_tc_tiling_on_sc=False),
        scratch_types=[
            pltpu.VMEM((c, ch), jnp.int32),
            pltpu.VMEM((ch, 16), jnp.float32),
            pltpu.VMEM((ch, 16), jnp.float32),
            pltpu.VMEM_SHARED((n_acc, 16), jnp.float32),
            pltpu.SemaphoreType.DMA,
            pltpu.SemaphoreType.DMA,
        ],
    )
    def k(msg_hbm, dst_hbm, zero_hbm, out_hbm, idx_v, buf0, buf1, acc_sh,
          sem0, sem1):
        cid = lax.axis_index("c")
        sid = lax.axis_index("s")
        wid = sid * _NC + cid
        base = wid * epw
        pltpu.sync_copy(dst_hbm.at[wid], idx_v)
        pltpu.sync_copy(zero_hbm.at[pl.ds(sid * rpt, rpt)],
                        acc_sh.at[pl.ds(sid * rpt, rpt)])
        plsc.subcore_barrier()

        pltpu.async_copy(msg_hbm.at[pl.ds(base, ch)], buf0, sem0)

        @pl.loop(0, (c - 1) // 2)
        def _(j):
            k0 = 2 * j
            pltpu.async_copy(msg_hbm.at[pl.ds(base + (k0 + 1) * ch, ch)],
                             buf1, sem1)
            pltpu.make_async_copy(msg_hbm.at[pl.ds(base + k0 * ch, ch)],
                                  buf0, sem0).wait()
            pltpu.sync_copy(buf0, acc_sh.at[idx_v.at[k0]], add=True)
            pltpu.async_copy(msg_hbm.at[pl.ds(base + (k0 + 2) * ch, ch)],
                             buf0, sem0)
            pltpu.make_async_copy(msg_hbm.at[pl.ds(base + (k0 + 1) * ch, ch)],
                                  buf1, sem1).wait()
            pltpu.sync_copy(buf1, acc_sh.at[idx_v.at[k0 + 1]], add=True)

        pltpu.make_async_copy(msg_hbm.at[pl.ds(base + (c - 1) * ch, ch)],
                              buf0, sem0).wait()
        pltpu.sync_copy(buf0, acc_sh.at[idx_v.at[c - 1]], add=True)

        plsc.subcore_barrier()
        pltpu.sync_copy(acc_sh.at[pl.ds(sid * rpt, rpt)],
                        out_hbm.at[cid, pl.ds(sid * rpt, rpt)])

    return k(msg, dst3, zeros)


def _edge_msgs(hsrc_p, attr_t, w1, b1, w2, b2, in_dim, blk, ep):
    kdim = w2.shape[1]
    h_out = kdim // in_dim
    slots = 128 // in_dim
    grid = ep // blk
    pr = blk // slots

    ii = np.arange(kdim) // h_out
    exps = np.zeros((slots, 128, kdim), np.float32)
    for p in range(slots):
        exps[p, p * in_dim + ii, np.arange(kdim)] = 1.0
    exps = jnp.asarray(exps, jnp.bfloat16)
    oo = np.arange(kdim) % h_out
    select = jnp.asarray(oo[:, None] == np.arange(h_out)[None, :],
                         jnp.bfloat16)

    def body(hsrc_ref, attr_ref, exp_ref, w1_ref, b1_ref, w2_ref, b2_ref,
             sel_ref, msg_ref):
        hb = hsrc_ref[...].astype(jnp.bfloat16)
        parts = [jnp.dot(hb, exp_ref[p], preferred_element_type=jnp.float32)
                 for p in range(slots)]
        hexp = jnp.concatenate(parts, axis=0)
        a = attr_ref[...].T.astype(jnp.bfloat16)
        z = jnp.dot(a, w1_ref[...],
                    preferred_element_type=jnp.float32) + b1_ref[...]
        u = jnp.maximum(z, 0.0).astype(jnp.bfloat16)
        wmat = jnp.dot(u, w2_ref[...],
                       preferred_element_type=jnp.float32) + b2_ref[...]
        prod = (hexp * wmat).astype(jnp.bfloat16)
        msg_ref[...] = jnp.dot(prod, sel_ref[...],
                               preferred_element_type=jnp.float32)

    full = lambda shape: pl.BlockSpec(shape, lambda i: (0,) * len(shape))
    return pl.pallas_call(
        body,
        grid=(grid,),
        in_specs=[
            pl.BlockSpec((pr, 128), lambda i: (i, 0)),
            pl.BlockSpec((4, blk), lambda i: (0, i)),
            full(exps.shape),
            full(w1.shape),
            full((1, kdim)),
            full(w2.shape),
            full((1, kdim)),
            full(select.shape),
        ],
        out_specs=pl.BlockSpec((blk, h_out), lambda i: (i, 0)),
        out_shape=jax.ShapeDtypeStruct((ep, h_out), jnp.float32),
    )(hsrc_p, attr_t, exps, w1.astype(jnp.bfloat16), b1.reshape(1, -1),
      w2.astype(jnp.bfloat16), b2.reshape(1, -1), select)


def _bn_layer(aggp, h, root, bias, gamma, beta, n):
    h_dim = root.shape[1]

    def body(agg_ref, h_ref, root_ref, bias_ref, gamma_ref, beta_ref,
             hout_ref, colsum_ref):
        agg = agg_ref[0, :n, :] + agg_ref[1, :n, :]
        hpre = agg + jnp.dot(h_ref[...], root_ref[...],
                             preferred_element_type=jnp.float32) + bias_ref[...]
        mean = jnp.mean(hpre, axis=0, keepdims=True)
        var = jnp.mean((hpre - mean) ** 2, axis=0, keepdims=True)
        hn = (hpre - mean) * lax.rsqrt(var + _EPS) * gamma_ref[...] + beta_ref[...]
        hout = jnp.maximum(hn, 0.0)
        hout_ref[...] = hout
        colsum_ref[...] = jnp.sum(hout, axis=0, keepdims=True)

    return pl.pallas_call(
        body,
        out_shape=(jax.ShapeDtypeStruct((n, h_dim), jnp.float32),
                   jax.ShapeDtypeStruct((1, h_dim), jnp.float32)),
    )(aggp, h, root, bias.reshape(1, -1), gamma.reshape(1, -1),
      beta.reshape(1, -1))


def _head(colsum_cat, jump_w, jump_b, reg_w1, reg_b1, reg_w2, reg_b2, n):
    def body(cs_ref, jw_ref, jb_ref, w1_ref, b1_ref, w2_ref, b2_ref, out_ref):
        pooled = jnp.dot(cs_ref[...], jw_ref[...],
                         preferred_element_type=jnp.float32) + n * jb_ref[...]
        r = jnp.maximum(jnp.dot(pooled, w1_ref[...],
                                preferred_element_type=jnp.float32)
                        + b1_ref[...], 0.0)
        out_ref[...] = jnp.dot(r, w2_ref[...],
                               preferred_element_type=jnp.float32) + b2_ref[...]

    return pl.pallas_call(
        body,
        out_shape=jax.ShapeDtypeStruct((1, 1), jnp.float32),
    )(colsum_cat, jump_w, jump_b.reshape(1, -1), reg_w1, reg_b1.reshape(1, -1),
      reg_w2, reg_b2.reshape(1, -1))


def kernel(x, edge_index, edge_attr, params):
    n = x.shape[0]
    e = edge_attr.shape[0]
    src = edge_index[0]
    dst = edge_index[1]

    ep = -(-e // 4096) * 4096
    epw = ep // _NW
    c = epw // _CH
    n_acc = ((n + 1 + _NS - 1) // _NS) * _NS

    src3 = jnp.pad(src, (0, ep - e)).reshape(_NW, c, _CH)
    dst_pad = jnp.pad(dst, (0, ep - e), constant_values=n)
    zeros_acc = jnp.zeros((n_acc, 16), jnp.float32)
    attr_t = jnp.pad(edge_attr.T, ((0, 0), (0, ep - e)))

    h = x
    colsums = []
    for li, lp in enumerate(params["layers"]):
        in_dim = h.shape[1]
        slots = 128 // in_dim
        blk = 1024 if li == 0 else 4096
        pr = blk // slots
        hsrc = _sc_gather(h, src3, ep)
        hsrc_p = hsrc.reshape(ep * in_dim // 128, 128)
        perm = lambda v: v.reshape(-1, ep // blk, pr, slots) \
                          .swapaxes(2, 3).reshape(v.shape[0], ep) \
            if v.ndim == 2 else \
            v.reshape(ep // blk, pr, slots).swapaxes(1, 2).reshape(ep)
        msg = _edge_msgs(hsrc_p, perm(attr_t), lp["w1"], lp["b1"], lp["w2"],
                         lp["b2"], in_dim, blk, ep)
        dst3 = perm(dst_pad).reshape(_NW, c, _CH)
        aggp = _sc_scatter(msg, dst3, zeros_acc, n_acc, ep)
        h, cs = _bn_layer(aggp, h, lp["root"], lp["bias"], lp["gamma"],
                          lp["beta"], n)
        colsums.append(cs)

    cs_cat = jnp.concatenate(colsums, axis=1)
    return _head(cs_cat, params["jump_w"], params["jump_b"],
                 params["reg_w1"], params["reg_b1"],
                 params["reg_w2"], params["reg_b2"], float(n))


# speedup vs baseline: 1.1354x; 1.0015x over previous
import functools

import jax
import jax.numpy as jnp
import numpy as np
from jax import lax
from jax.experimental import pallas as pl
from jax.experimental.pallas import tpu as pltpu
from jax.experimental.pallas import tpu_sc as plsc

_EPS = 1e-5
_NC = 2
_NS = 16
_NW = _NC * _NS
_CH = 128


def _worker_mesh():
    return plsc.VectorSubcoreMesh(core_axis_name="c", subcore_axis_name="s",
                                  num_cores=_NC, num_subcores=_NS)


def _sc_gather(table, idx3, ep):
    nw, c, ch = idx3.shape
    d = table.shape[1]
    epw = ep // _NW

    @functools.partial(
        pl.kernel,
        out_type=jax.ShapeDtypeStruct((ep, d), jnp.float32),
        mesh=_worker_mesh(),
        compiler_params=pltpu.CompilerParams(use_tc_tiling_on_sc=False),
        scratch_types=[
            pltpu.VMEM((c, ch), jnp.int32),
            pltpu.VMEM((ch, d), jnp.float32),
            pltpu.VMEM((ch, d), jnp.float32),
            pltpu.SemaphoreType.DMA,
            pltpu.SemaphoreType.DMA,
        ],
    )
    def k(table_hbm, idx_hbm, out_hbm, idx_v, u0, u1, sg0, sg1):
        cid = lax.axis_index("c")
        sid = lax.axis_index("s")
        wid = sid * _NC + cid
        base = wid * epw
        pltpu.sync_copy(idx_hbm.at[wid], idx_v)

        def start(k_, u, sg):
            pltpu.async_copy(table_hbm.at[idx_v.at[k_]], u, sg)

        def finish(k_, u, sg):
            pltpu.make_async_copy(table_hbm.at[idx_v.at[k_]], u, sg).wait()
            pltpu.sync_copy(u, out_hbm.at[pl.ds(base + k_ * ch, ch)])

        start(0, u0, sg0)

        @pl.loop(0, (c - 1) // 2)
        def _(j):
            k0 = 2 * j
            start(k0 + 1, u1, sg1)
            finish(k0, u0, sg0)
            start(k0 + 2, u0, sg0)
            finish(k0 + 1, u1, sg1)

        finish(c - 1, u0, sg0)

    return k(table, idx3)


def _sc_scatter(msg, dst3, zeros, n_acc, ep):
    nw, c, ch = dst3.shape
    epw = ep // _NW
    rpt = n_acc // _NS

    @functools.partial(
        pl.kernel,
        out_type=jax.ShapeDtypeStruct((_NC, n_acc, 16), jnp.float32),
        mesh=_worker_mesh(),
        compiler_params=pltpu.CompilerParams(use_tc_tiling_on_sc=False),
        scratch_types=[
            pltpu.VMEM((c, ch), jnp.int32),
            pltpu.VMEM((ch, 16), jnp.float32),
            pltpu.VMEM((ch, 16), jnp.float32),
            pltpu.VMEM_SHARED((n_acc, 16), jnp.float32),
            pltpu.SemaphoreType.DMA,
            pltpu.SemaphoreType.DMA,
        ],
    )
    def k(msg_hbm, dst_hbm, zero_hbm, out_hbm, idx_v, buf0, buf1, acc_sh,
          sem0, sem1):
        cid = lax.axis_index("c")
        sid = lax.axis_index("s")
        wid = sid * _NC + cid
        base = wid * epw
        pltpu.sync_copy(dst_hbm.at[wid], idx_v)
        pltpu.sync_copy(zero_hbm.at[pl.ds(sid * rpt, rpt)],
                        acc_sh.at[pl.ds(sid * rpt, rpt)])
        plsc.subcore_barrier()

        pltpu.async_copy(msg_hbm.at[pl.ds(base, ch)], buf0, sem0)

        @pl.loop(0, (c - 1) // 2)
        def _(j):
            k0 = 2 * j
            pltpu.async_copy(msg_hbm.at[pl.ds(base + (k0 + 1) * ch, ch)],
                             buf1, sem1)
            pltpu.make_async_copy(msg_hbm.at[pl.ds(base + k0 * ch, ch)],
                                  buf0, sem0).wait()
            pltpu.sync_copy(buf0, acc_sh.at[idx_v.at[k0]], add=True)
            pltpu.async_copy(msg_hbm.at[pl.ds(base + (k0 + 2) * ch, ch)],
                             buf0, sem0)
            pltpu.make_async_copy(msg_hbm.at[pl.ds(base + (k0 + 1) * ch, ch)],
                                  buf1, sem1).wait()
            pltpu.sync_copy(buf1, acc_sh.at[idx_v.at[k0 + 1]], add=True)

        pltpu.make_async_copy(msg_hbm.at[pl.ds(base + (c - 1) * ch, ch)],
                              buf0, sem0).wait()
        pltpu.sync_copy(buf0, acc_sh.at[idx_v.at[c - 1]], add=True)

        plsc.subcore_barrier()
        pltpu.sync_copy(acc_sh.at[pl.ds(sid * rpt, rpt)],
                        out_hbm.at[cid, pl.ds(sid * rpt, rpt)])

    return k(msg, dst3, zeros)


def _edge_msgs(hsrc_p, attr_t, w1, b1, w2, b2, in_dim, blk, ep):
    kdim = w2.shape[1]
    h_out = kdim // in_dim
    slots = 128 // in_dim
    grid = ep // blk
    pr = blk // slots

    ii = np.arange(kdim) // h_out
    exps = np.zeros((slots, 128, kdim), np.float32)
    for p in range(slots):
        exps[p, p * in_dim + ii, np.arange(kdim)] = 1.0
    exps = jnp.asarray(exps, jnp.bfloat16)
    oo = np.arange(kdim) % h_out
    select = jnp.asarray(oo[:, None] == np.arange(h_out)[None, :],
                         jnp.float32)

    def body(hsrc_ref, attr_ref, exp_ref, w1_ref, b1_ref, w2_ref, b2_ref,
             sel_ref, msg_ref):
        hb = hsrc_ref[...].astype(jnp.bfloat16)
        parts = [jnp.dot(hb, exp_ref[p], preferred_element_type=jnp.float32)
                 for p in range(slots)]
        hexp = jnp.concatenate(parts, axis=0)
        a = attr_ref[...].T.astype(jnp.bfloat16)
        z = jnp.dot(a, w1_ref[...],
                    preferred_element_type=jnp.float32) + b1_ref[...]
        u = jnp.maximum(z, 0.0).astype(jnp.bfloat16)
        wmat = jnp.dot(u, w2_ref[...],
                       preferred_element_type=jnp.float32) + b2_ref[...]
        prod = hexp * wmat
        msg_ref[...] = jnp.dot(prod, sel_ref[...],
                               preferred_element_type=jnp.float32)

    full = lambda shape: pl.BlockSpec(shape, lambda i: (0,) * len(shape))
    return pl.pallas_call(
        body,
        grid=(grid,),
        in_specs=[
            pl.BlockSpec((pr, 128), lambda i: (i, 0)),
            pl.BlockSpec((4, blk), lambda i: (0, i)),
            full(exps.shape),
            full(w1.shape),
            full((1, kdim)),
            full(w2.shape),
            full((1, kdim)),
            full(select.shape),
        ],
        out_specs=pl.BlockSpec((blk, h_out), lambda i: (i, 0)),
        out_shape=jax.ShapeDtypeStruct((ep, h_out), jnp.float32),
    )(hsrc_p, attr_t, exps, w1.astype(jnp.bfloat16), b1.reshape(1, -1),
      w2.astype(jnp.bfloat16), b2.reshape(1, -1), select)


def _bn_layer(aggp, h, root, bias, gamma, beta, n):
    h_dim = root.shape[1]

    def body(agg_ref, h_ref, root_ref, bias_ref, gamma_ref, beta_ref,
             hout_ref, colsum_ref):
        agg = agg_ref[0, :n, :] + agg_ref[1, :n, :]
        hpre = agg + jnp.dot(h_ref[...], root_ref[...],
                             preferred_element_type=jnp.float32) + bias_ref[...]
        mean = jnp.mean(hpre, axis=0, keepdims=True)
        var = jnp.mean((hpre - mean) ** 2, axis=0, keepdims=True)
        hn = (hpre - mean) * lax.rsqrt(var + _EPS) * gamma_ref[...] + beta_ref[...]
        hout = jnp.maximum(hn, 0.0)
        hout_ref[...] = hout
        colsum_ref[...] = jnp.sum(hout, axis=0, keepdims=True)

    return pl.pallas_call(
        body,
        out_shape=(jax.ShapeDtypeStruct((n, h_dim), jnp.float32),
                   jax.ShapeDtypeStruct((1, h_dim), jnp.float32)),
    )(aggp, h, root, bias.reshape(1, -1), gamma.reshape(1, -1),
      beta.reshape(1, -1))


def _head(colsum_cat, jump_w, jump_b, reg_w1, reg_b1, reg_w2, reg_b2, n):
    def body(cs_ref, jw_ref, jb_ref, w1_ref, b1_ref, w2_ref, b2_ref, out_ref):
        pooled = jnp.dot(cs_ref[...], jw_ref[...],
                         preferred_element_type=jnp.float32) + n * jb_ref[...]
        r = jnp.maximum(jnp.dot(pooled, w1_ref[...],
                                preferred_element_type=jnp.float32)
                        + b1_ref[...], 0.0)
        out_ref[...] = jnp.dot(r, w2_ref[...],
                               preferred_element_type=jnp.float32) + b2_ref[...]

    return pl.pallas_call(
        body,
        out_shape=jax.ShapeDtypeStruct((1, 1), jnp.float32),
    )(colsum_cat, jump_w, jump_b.reshape(1, -1), reg_w1, reg_b1.reshape(1, -1),
      reg_w2, reg_b2.reshape(1, -1))


def kernel(x, edge_index, edge_attr, params):
    n = x.shape[0]
    e = edge_attr.shape[0]
    src = edge_index[0]
    dst = edge_index[1]

    ep = -(-e // 4096) * 4096
    epw = ep // _NW
    c = epw // _CH
    n_acc = ((n + 1 + _NS - 1) // _NS) * _NS

    src3 = jnp.pad(src, (0, ep - e)).reshape(_NW, c, _CH)
    dst_pad = jnp.pad(dst, (0, ep - e), constant_values=n)
    zeros_acc = jnp.zeros((n_acc, 16), jnp.float32)
    attr_t = jnp.pad(edge_attr.T, ((0, 0), (0, ep - e)))

    h = x
    colsums = []
    for li, lp in enumerate(params["layers"]):
        in_dim = h.shape[1]
        slots = 128 // in_dim
        blk = 1024 if li == 0 else 4096
        pr = blk // slots
        hsrc = _sc_gather(h, src3, ep)
        hsrc_p = hsrc.reshape(ep * in_dim // 128, 128)
        perm = lambda v: v.reshape(-1, ep // blk, pr, slots) \
                          .swapaxes(2, 3).reshape(v.shape[0], ep) \
            if v.ndim == 2 else \
            v.reshape(ep // blk, pr, slots).swapaxes(1, 2).reshape(ep)
        msg = _edge_msgs(hsrc_p, perm(attr_t), lp["w1"], lp["b1"], lp["w2"],
                         lp["b2"], in_dim, blk, ep)
        dst3 = perm(dst_pad).reshape(_NW, c, _CH)
        aggp = _sc_scatter(msg, dst3, zeros_acc, n_acc, ep)
        h, cs = _bn_layer(aggp, h, lp["root"], lp["bias"], lp["gamma"],
                          lp["beta"], n)
        colsums.append(cs)

    cs_cat = jnp.concatenate(colsums, axis=1)
    return _head(cs_cat, params["jump_w"], params["jump_b"],
                 params["reg_w1"], params["reg_b1"],
                 params["reg_w2"], params["reg_b2"], float(n))


# speedup vs baseline: 1.2167x; 1.0716x over previous
import functools

import jax
import jax.numpy as jnp
import numpy as np
from jax import lax
from jax.experimental import pallas as pl
from jax.experimental.pallas import tpu as pltpu
from jax.experimental.pallas import tpu_sc as plsc

_EPS = 1e-5
_NC = 2
_NS = 16
_NW = _NC * _NS
_CH = 128


def _worker_mesh():
    return plsc.VectorSubcoreMesh(core_axis_name="c", subcore_axis_name="s",
                                  num_cores=_NC, num_subcores=_NS)


def _sc_gather(table, idx3, ep):
    nw, c, ch = idx3.shape
    d = table.shape[1]
    epw = ep // _NW

    @functools.partial(
        pl.kernel,
        out_type=jax.ShapeDtypeStruct((ep, d), jnp.float32),
        mesh=_worker_mesh(),
        compiler_params=pltpu.CompilerParams(use_tc_tiling_on_sc=False),
        scratch_types=[
            pltpu.VMEM((c, ch), jnp.int32),
            pltpu.VMEM((ch, d), jnp.float32),
            pltpu.VMEM((ch, d), jnp.float32),
            pltpu.SemaphoreType.DMA,
            pltpu.SemaphoreType.DMA,
        ],
    )
    def k(table_hbm, idx_hbm, out_hbm, idx_v, u0, u1, sg0, sg1):
        cid = lax.axis_index("c")
        sid = lax.axis_index("s")
        wid = sid * _NC + cid
        base = wid * epw
        pltpu.sync_copy(idx_hbm.at[wid], idx_v)

        def start(k_, u, sg):
            pltpu.async_copy(table_hbm.at[idx_v.at[k_]], u, sg)

        def finish(k_, u, sg):
            pltpu.make_async_copy(table_hbm.at[idx_v.at[k_]], u, sg).wait()
            pltpu.sync_copy(u, out_hbm.at[pl.ds(base + k_ * ch, ch)])

        start(0, u0, sg0)

        @pl.loop(0, (c - 1) // 2)
        def _(j):
            k0 = 2 * j
            start(k0 + 1, u1, sg1)
            finish(k0, u0, sg0)
            start(k0 + 2, u0, sg0)
            finish(k0 + 1, u1, sg1)

        finish(c - 1, u0, sg0)

    return k(table, idx3)


def _sc_scatter(msg, dst3, zeros, n_acc, ep):
    nw, c, ch = dst3.shape
    epw = ep // _NW
    rpt = n_acc // _NS

    @functools.partial(
        pl.kernel,
        out_type=jax.ShapeDtypeStruct((_NC, n_acc, 16), jnp.float32),
        mesh=_worker_mesh(),
        compiler_params=pltpu.CompilerParams(use_tc_tiling_on_sc=False),
        scratch_types=[
            pltpu.VMEM((c, ch), jnp.int32),
            pltpu.VMEM((ch, 16), jnp.float32),
            pltpu.VMEM((ch, 16), jnp.float32),
            pltpu.VMEM_SHARED((n_acc, 16), jnp.float32),
            pltpu.SemaphoreType.DMA,
            pltpu.SemaphoreType.DMA,
        ],
    )
    def k(msg_hbm, dst_hbm, zero_hbm, out_hbm, idx_v, buf0, buf1, acc_sh,
          sem0, sem1):
        cid = lax.axis_index("c")
        sid = lax.axis_index("s")
        wid = sid * _NC + cid
        base = wid * epw
        pltpu.sync_copy(dst_hbm.at[wid], idx_v)
        pltpu.sync_copy(zero_hbm.at[pl.ds(sid * rpt, rpt)],
                        acc_sh.at[pl.ds(sid * rpt, rpt)])
        plsc.subcore_barrier()

        pltpu.async_copy(msg_hbm.at[pl.ds(base, ch)], buf0, sem0)

        @pl.loop(0, (c - 1) // 2)
        def _(j):
            k0 = 2 * j
            pltpu.async_copy(msg_hbm.at[pl.ds(base + (k0 + 1) * ch, ch)],
                             buf1, sem1)
            pltpu.make_async_copy(msg_hbm.at[pl.ds(base + k0 * ch, ch)],
                                  buf0, sem0).wait()
            pltpu.sync_copy(buf0, acc_sh.at[idx_v.at[k0]], add=True)
            pltpu.async_copy(msg_hbm.at[pl.ds(base + (k0 + 2) * ch, ch)],
                             buf0, sem0)
            pltpu.make_async_copy(msg_hbm.at[pl.ds(base + (k0 + 1) * ch, ch)],
                                  buf1, sem1).wait()
            pltpu.sync_copy(buf1, acc_sh.at[idx_v.at[k0 + 1]], add=True)

        pltpu.make_async_copy(msg_hbm.at[pl.ds(base + (c - 1) * ch, ch)],
                              buf0, sem0).wait()
        pltpu.sync_copy(buf0, acc_sh.at[idx_v.at[c - 1]], add=True)

        plsc.subcore_barrier()
        pltpu.sync_copy(acc_sh.at[pl.ds(sid * rpt, rpt)],
                        out_hbm.at[cid, pl.ds(sid * rpt, rpt)])

    return k(msg, dst3, zeros)


def _edge_msgs(hsrc_p, attr_t, w1, b1, w2, b2, in_dim, blk, ep):
    kdim = w2.shape[1]
    h_out = kdim // in_dim
    slots = 128 // in_dim
    grid = ep // blk
    pr = blk // slots

    ii = np.arange(kdim) // h_out
    exps = np.zeros((slots, 128, kdim), np.float32)
    for p in range(slots):
        exps[p, p * in_dim + ii, np.arange(kdim)] = 1.0
    exps = jnp.asarray(exps, jnp.bfloat16)
    oo = np.arange(kdim) % h_out
    select = jnp.asarray(oo[:, None] == np.arange(h_out)[None, :],
                         jnp.float32)

    def body(hsrc_ref, attr_ref, exp_ref, w1_ref, b1_ref, w2_ref, b2_ref,
             sel_ref, msg_ref):
        hb = hsrc_ref[...].astype(jnp.bfloat16)
        parts = [jnp.dot(hb, exp_ref[p], preferred_element_type=jnp.float32)
                 for p in range(slots)]
        hexp = jnp.concatenate(parts, axis=0)
        a = attr_ref[...].T.astype(jnp.bfloat16)
        z = jnp.dot(a, w1_ref[...],
                    preferred_element_type=jnp.float32) + b1_ref[...]
        u = jnp.maximum(z, 0.0).astype(jnp.bfloat16)
        wmat = jnp.dot(u, w2_ref[...],
                       preferred_element_type=jnp.float32) + b2_ref[...]
        prod = hexp * wmat
        msg_ref[...] = jnp.dot(prod, sel_ref[...],
                               preferred_element_type=jnp.float32)

    full = lambda shape: pl.BlockSpec(shape, lambda i: (0,) * len(shape))
    return pl.pallas_call(
        body,
        grid=(grid,),
        in_specs=[
            pl.BlockSpec((pr, 128), lambda i: (i, 0)),
            pl.BlockSpec((4, blk), lambda i: (0, i)),
            full(exps.shape),
            full(w1.shape),
            full((1, kdim)),
            full(w2.shape),
            full((1, kdim)),
            full(select.shape),
        ],
        out_specs=pl.BlockSpec((blk, h_out), lambda i: (i, 0)),
        out_shape=jax.ShapeDtypeStruct((ep, h_out), jnp.float32),
    )(hsrc_p, attr_t, exps, w1.astype(jnp.bfloat16), b1.reshape(1, -1),
      w2.astype(jnp.bfloat16), b2.reshape(1, -1), select)


def _bn_layer(aggp, h, root, bias, gamma, beta, n):
    h_dim = root.shape[1]

    def body(agg_ref, h_ref, root_ref, bias_ref, gamma_ref, beta_ref,
             hout_ref, colsum_ref):
        agg = agg_ref[0, :n, :] + agg_ref[1, :n, :]
        hpre = agg + jnp.dot(h_ref[...], root_ref[...],
                             preferred_element_type=jnp.float32) + bias_ref[...]
        mean = jnp.mean(hpre, axis=0, keepdims=True)
        var = jnp.mean((hpre - mean) ** 2, axis=0, keepdims=True)
        hn = (hpre - mean) * lax.rsqrt(var + _EPS) * gamma_ref[...] + beta_ref[...]
        hout = jnp.maximum(hn, 0.0)
        hout_ref[...] = hout
        colsum_ref[...] = jnp.sum(hout, axis=0, keepdims=True)

    return pl.pallas_call(
        body,
        out_shape=(jax.ShapeDtypeStruct((n, h_dim), jnp.float32),
                   jax.ShapeDtypeStruct((1, h_dim), jnp.float32)),
    )(aggp, h, root, bias.reshape(1, -1), gamma.reshape(1, -1),
      beta.reshape(1, -1))


def _head(colsum_cat, jump_w, jump_b, reg_w1, reg_b1, reg_w2, reg_b2, n):
    def body(cs_ref, jw_ref, jb_ref, w1_ref, b1_ref, w2_ref, b2_ref, out_ref):
        pooled = jnp.dot(cs_ref[...], jw_ref[...],
                         preferred_element_type=jnp.float32) + n * jb_ref[...]
        r = jnp.maximum(jnp.dot(pooled, w1_ref[...],
                                preferred_element_type=jnp.float32)
                        + b1_ref[...], 0.0)
        out_ref[...] = jnp.dot(r, w2_ref[...],
                               preferred_element_type=jnp.float32) + b2_ref[...]

    return pl.pallas_call(
        body,
        out_shape=jax.ShapeDtypeStruct((1, 1), jnp.float32),
    )(colsum_cat, jump_w, jump_b.reshape(1, -1), reg_w1, reg_b1.reshape(1, -1),
      reg_w2, reg_b2.reshape(1, -1))


def kernel(x, edge_index, edge_attr, params):
    n = x.shape[0]
    e = edge_attr.shape[0]
    src = edge_index[0]
    dst = edge_index[1]

    ep = -(-e // 4096) * 4096
    epw = ep // _NW
    c = epw // _CH
    n_acc = ((n + 1 + _NS - 1) // _NS) * _NS

    src_pad = jnp.pad(src, (0, ep - e))
    dst3 = jnp.pad(dst, (0, ep - e),
                   constant_values=n).reshape(_NW, c, _CH)
    zeros_acc = jnp.zeros((n_acc, 16), jnp.float32)
    attr_t = jnp.pad(edge_attr.T, ((0, 0), (0, ep - e)))

    h = x
    colsums = []
    for li, lp in enumerate(params["layers"]):
        in_dim = h.shape[1]
        slots = 128 // in_dim
        blk = 1024 if li == 0 else 4096
        pr = blk // slots
        src3 = src_pad.reshape(ep // blk, slots, pr) \
                      .swapaxes(1, 2).reshape(_NW, c, _CH)
        hsrc = _sc_gather(h, src3, ep)
        hsrc_p = hsrc.reshape(ep * in_dim // 128, 128)
        msg = _edge_msgs(hsrc_p, attr_t, lp["w1"], lp["b1"], lp["w2"],
                         lp["b2"], in_dim, blk, ep)
        aggp = _sc_scatter(msg, dst3, zeros_acc, n_acc, ep)
        h, cs = _bn_layer(aggp, h, lp["root"], lp["bias"], lp["gamma"],
                          lp["beta"], n)
        colsums.append(cs)

    cs_cat = jnp.concatenate(colsums, axis=1)
    return _head(cs_cat, params["jump_w"], params["jump_b"],
                 params["reg_w1"], params["reg_b1"],
                 params["reg_w2"], params["reg_b2"], float(n))


# speedup vs baseline: 1.3434x; 1.1042x over previous
import functools

import jax
import jax.numpy as jnp
import numpy as np
from jax import lax
from jax.experimental import pallas as pl
from jax.experimental.pallas import tpu as pltpu
from jax.experimental.pallas import tpu_sc as plsc

_EPS = 1e-5
_NC = 2
_NS = 16
_NW = _NC * _NS
_CH = 128


def _worker_mesh():
    return plsc.VectorSubcoreMesh(core_axis_name="c", subcore_axis_name="s",
                                  num_cores=_NC, num_subcores=_NS)


def _sc_gather(table, idx3, ep):
    nw, c, ch = idx3.shape
    d = table.shape[1]
    epw = ep // _NW

    @functools.partial(
        pl.kernel,
        out_type=jax.ShapeDtypeStruct((ep, d), jnp.float32),
        mesh=_worker_mesh(),
        compiler_params=pltpu.CompilerParams(use_tc_tiling_on_sc=False),
        scratch_types=[
            pltpu.VMEM((c, ch), jnp.int32),
            pltpu.VMEM((ch, d), jnp.float32),
            pltpu.VMEM((ch, d), jnp.float32),
            pltpu.SemaphoreType.DMA,
            pltpu.SemaphoreType.DMA,
        ],
    )
    def k(table_hbm, idx_hbm, out_hbm, idx_v, u0, u1, sg0, sg1):
        cid = lax.axis_index("c")
        sid = lax.axis_index("s")
        wid = sid * _NC + cid
        base = wid * epw
        pltpu.sync_copy(idx_hbm.at[wid], idx_v)

        def start(k_, u, sg):
            pltpu.async_copy(table_hbm.at[idx_v.at[k_]], u, sg)

        def finish(k_, u, sg):
            pltpu.make_async_copy(table_hbm.at[idx_v.at[k_]], u, sg).wait()
            pltpu.sync_copy(u, out_hbm.at[pl.ds(base + k_ * ch, ch)])

        start(0, u0, sg0)

        @pl.loop(0, (c - 1) // 2)
        def _(j):
            k0 = 2 * j
            start(k0 + 1, u1, sg1)
            finish(k0, u0, sg0)
            start(k0 + 2, u0, sg0)
            finish(k0 + 1, u1, sg1)

        finish(c - 1, u0, sg0)

    return k(table, idx3)


def _sc_scatter(msg, dst3, zeros, n_acc, ep):
    nw, c, ch = dst3.shape
    epw = ep // _NW
    rpt = n_acc // _NS

    @functools.partial(
        pl.kernel,
        out_type=jax.ShapeDtypeStruct((_NC, n_acc, 16), jnp.float32),
        mesh=_worker_mesh(),
        compiler_params=pltpu.CompilerParams(use_tc_tiling_on_sc=False),
        scratch_types=[
            pltpu.VMEM((c, ch), jnp.int32),
            pltpu.VMEM((ch, 16), jnp.float32),
            pltpu.VMEM((ch, 16), jnp.float32),
            pltpu.VMEM_SHARED((n_acc, 16), jnp.float32),
            pltpu.SemaphoreType.DMA,
            pltpu.SemaphoreType.DMA,
        ],
    )
    def k(msg_hbm, dst_hbm, zero_hbm, out_hbm, idx_v, buf0, buf1, acc_sh,
          sem0, sem1):
        cid = lax.axis_index("c")
        sid = lax.axis_index("s")
        wid = sid * _NC + cid
        base = wid * epw
        pltpu.sync_copy(dst_hbm.at[wid], idx_v)
        pltpu.sync_copy(zero_hbm.at[pl.ds(sid * rpt, rpt)],
                        acc_sh.at[pl.ds(sid * rpt, rpt)])
        plsc.subcore_barrier()

        pltpu.async_copy(msg_hbm.at[pl.ds(base, ch)], buf0, sem0)

        @pl.loop(0, (c - 1) // 2)
        def _(j):
            k0 = 2 * j
            pltpu.async_copy(msg_hbm.at[pl.ds(base + (k0 + 1) * ch, ch)],
                             buf1, sem1)
            pltpu.make_async_copy(msg_hbm.at[pl.ds(base + k0 * ch, ch)],
                                  buf0, sem0).wait()
            pltpu.sync_copy(buf0, acc_sh.at[idx_v.at[k0]], add=True)
            pltpu.async_copy(msg_hbm.at[pl.ds(base + (k0 + 2) * ch, ch)],
                             buf0, sem0)
            pltpu.make_async_copy(msg_hbm.at[pl.ds(base + (k0 + 1) * ch, ch)],
                                  buf1, sem1).wait()
            pltpu.sync_copy(buf1, acc_sh.at[idx_v.at[k0 + 1]], add=True)

        pltpu.make_async_copy(msg_hbm.at[pl.ds(base + (c - 1) * ch, ch)],
                              buf0, sem0).wait()
        pltpu.sync_copy(buf0, acc_sh.at[idx_v.at[c - 1]], add=True)

        plsc.subcore_barrier()
        pltpu.sync_copy(acc_sh.at[pl.ds(sid * rpt, rpt)],
                        out_hbm.at[cid, pl.ds(sid * rpt, rpt)])

    return k(msg, dst3, zeros)


def _edge_msgs(hsrc_p, attr_t, w1, b1, w2, b2, in_dim, blk, ep):
    kdim = w2.shape[1]
    h_out = kdim // in_dim
    slots = 128 // in_dim
    grid = ep // blk
    pr = blk // slots

    ii = np.arange(kdim) // h_out
    exps = np.zeros((slots, 128, kdim), np.float32)
    for p in range(slots):
        exps[p, p * in_dim + ii, np.arange(kdim)] = 1.0
    exps = jnp.asarray(exps, jnp.bfloat16)
    oo = np.arange(kdim) % h_out
    select = jnp.asarray(oo[:, None] == np.arange(h_out)[None, :],
                         jnp.float32)

    def body(hsrc_ref, attr_ref, exp_ref, w1_ref, b1_ref, w2_ref, b2_ref,
             sel_ref, msg_ref):
        hb = hsrc_ref[...].astype(jnp.bfloat16)
        parts = [jnp.dot(hb, exp_ref[p], preferred_element_type=jnp.float32)
                 for p in range(slots)]
        hexp = jnp.concatenate(parts, axis=0)
        a = attr_ref[...].T.astype(jnp.bfloat16)
        z = jnp.dot(a, w1_ref[...],
                    preferred_element_type=jnp.float32) + b1_ref[...]
        u = jnp.maximum(z, 0.0).astype(jnp.bfloat16)
        wmat = jnp.dot(u, w2_ref[...],
                       preferred_element_type=jnp.float32) + b2_ref[...]
        prod = hexp * wmat
        msg = jnp.dot(prod, sel_ref[...], preferred_element_type=jnp.float32)
        s8 = blk // 8
        msg_ref[...] = jnp.concatenate(
            [msg[j * s8:(j + 1) * s8, :] for j in range(8)], axis=1)

    full = lambda shape: pl.BlockSpec(shape, lambda i: (0,) * len(shape))
    return pl.pallas_call(
        body,
        grid=(grid,),
        in_specs=[
            pl.BlockSpec((pr, 128), lambda i: (i, 0)),
            pl.BlockSpec((4, blk), lambda i: (0, i)),
            full(exps.shape),
            full(w1.shape),
            full((1, kdim)),
            full(w2.shape),
            full((1, kdim)),
            full(select.shape),
        ],
        out_specs=pl.BlockSpec((blk // 8, 128), lambda i: (i, 0)),
        out_shape=jax.ShapeDtypeStruct((ep * h_out // 128, 128), jnp.float32),
    )(hsrc_p, attr_t, exps, w1.astype(jnp.bfloat16), b1.reshape(1, -1),
      w2.astype(jnp.bfloat16), b2.reshape(1, -1), select)


def _bn_layer(aggp, h, root, bias, gamma, beta, n):
    h_dim = root.shape[1]

    def body(agg_ref, h_ref, root_ref, bias_ref, gamma_ref, beta_ref,
             hout_ref, colsum_ref):
        agg = agg_ref[0, :n, :] + agg_ref[1, :n, :]
        hpre = agg + jnp.dot(h_ref[...], root_ref[...],
                             preferred_element_type=jnp.float32) + bias_ref[...]
        mean = jnp.mean(hpre, axis=0, keepdims=True)
        var = jnp.mean((hpre - mean) ** 2, axis=0, keepdims=True)
        hn = (hpre - mean) * lax.rsqrt(var + _EPS) * gamma_ref[...] + beta_ref[...]
        hout = jnp.maximum(hn, 0.0)
        hout_ref[...] = hout
        colsum_ref[...] = jnp.sum(hout, axis=0, keepdims=True)

    return pl.pallas_call(
        body,
        out_shape=(jax.ShapeDtypeStruct((n, h_dim), jnp.float32),
                   jax.ShapeDtypeStruct((1, h_dim), jnp.float32)),
    )(aggp, h, root, bias.reshape(1, -1), gamma.reshape(1, -1),
      beta.reshape(1, -1))


def _head(colsum_cat, jump_w, jump_b, reg_w1, reg_b1, reg_w2, reg_b2, n):
    def body(cs_ref, jw_ref, jb_ref, w1_ref, b1_ref, w2_ref, b2_ref, out_ref):
        pooled = jnp.dot(cs_ref[...], jw_ref[...],
                         preferred_element_type=jnp.float32) + n * jb_ref[...]
        r = jnp.maximum(jnp.dot(pooled, w1_ref[...],
                                preferred_element_type=jnp.float32)
                        + b1_ref[...], 0.0)
        out_ref[...] = jnp.dot(r, w2_ref[...],
                               preferred_element_type=jnp.float32) + b2_ref[...]

    return pl.pallas_call(
        body,
        out_shape=jax.ShapeDtypeStruct((1, 1), jnp.float32),
    )(colsum_cat, jump_w, jump_b.reshape(1, -1), reg_w1, reg_b1.reshape(1, -1),
      reg_w2, reg_b2.reshape(1, -1))


def kernel(x, edge_index, edge_attr, params):
    n = x.shape[0]
    e = edge_attr.shape[0]
    src = edge_index[0]
    dst = edge_index[1]

    ep = -(-e // 4096) * 4096
    epw = ep // _NW
    c = epw // _CH
    n_acc = ((n + 1 + _NS - 1) // _NS) * _NS

    src_pad = jnp.pad(src, (0, ep - e))
    dst_pad = jnp.pad(dst, (0, ep - e), constant_values=n)
    zeros_acc = jnp.zeros((n_acc, 16), jnp.float32)
    attr_t = jnp.pad(edge_attr.T, ((0, 0), (0, ep - e)))

    h = x
    colsums = []
    for li, lp in enumerate(params["layers"]):
        in_dim = h.shape[1]
        slots = 128 // in_dim
        blk = 1024 if li == 0 else 4096
        pr = blk // slots
        src3 = src_pad.reshape(ep // blk, slots, pr) \
                      .swapaxes(1, 2).reshape(_NW, c, _CH)
        hsrc = _sc_gather(h, src3, ep)
        hsrc_p = hsrc.reshape(ep * in_dim // 128, 128)
        msg_p = _edge_msgs(hsrc_p, attr_t, lp["w1"], lp["b1"], lp["w2"],
                           lp["b2"], in_dim, blk, ep)
        dst3 = dst_pad.reshape(ep // blk, 8, blk // 8)                       .swapaxes(1, 2).reshape(_NW, c, _CH)
        aggp = _sc_scatter(msg_p.reshape(ep, 16), dst3, zeros_acc, n_acc, ep)
        h, cs = _bn_layer(aggp, h, lp["root"], lp["bias"], lp["gamma"],
                          lp["beta"], n)
        colsums.append(cs)

    cs_cat = jnp.concatenate(colsums, axis=1)
    return _head(cs_cat, params["jump_w"], params["jump_b"],
                 params["reg_w1"], params["reg_b1"],
                 params["reg_w2"], params["reg_b2"], float(n))


# speedup vs baseline: 1.3504x; 1.0052x over previous
import functools

import jax
import jax.numpy as jnp
import numpy as np
from jax import lax
from jax.experimental import pallas as pl
from jax.experimental.pallas import tpu as pltpu
from jax.experimental.pallas import tpu_sc as plsc

_EPS = 1e-5
_NC = 2
_NS = 16
_NW = _NC * _NS
_CH = 128


def _worker_mesh():
    return plsc.VectorSubcoreMesh(core_axis_name="c", subcore_axis_name="s",
                                  num_cores=_NC, num_subcores=_NS)


def _sc_gather(table, idx3, ep):
    nw, c, ch = idx3.shape
    d = table.shape[1]
    epw = ep // _NW

    @functools.partial(
        pl.kernel,
        out_type=jax.ShapeDtypeStruct((ep, d), jnp.float32),
        mesh=_worker_mesh(),
        compiler_params=pltpu.CompilerParams(use_tc_tiling_on_sc=False),
        scratch_types=[
            pltpu.VMEM((c, ch), jnp.int32),
            pltpu.VMEM((ch, d), jnp.float32),
            pltpu.VMEM((ch, d), jnp.float32),
            pltpu.VMEM((ch, d), jnp.float32),
            pltpu.VMEM((ch, d), jnp.float32),
            pltpu.SemaphoreType.DMA,
            pltpu.SemaphoreType.DMA,
            pltpu.SemaphoreType.DMA,
            pltpu.SemaphoreType.DMA,
        ],
    )
    def k(table_hbm, idx_hbm, out_hbm, idx_v, u0, u1, u2, u3,
          sg0, sg1, sg2, sg3):
        cid = lax.axis_index("c")
        sid = lax.axis_index("s")
        wid = sid * _NC + cid
        base = wid * epw
        pltpu.sync_copy(idx_hbm.at[wid], idx_v)
        us = (u0, u1, u2, u3)
        sgs = (sg0, sg1, sg2, sg3)

        for t in range(4):
            pltpu.async_copy(table_hbm.at[idx_v.at[t]], us[t], sgs[t])

        @pl.loop(0, (c - 5) // 4)
        def _(j):
            for t in range(4):
                k_ = 4 * j + t
                pltpu.make_async_copy(table_hbm.at[idx_v.at[k_]],
                                      us[t], sgs[t]).wait()
                pltpu.sync_copy(us[t], out_hbm.at[pl.ds(base + k_ * ch, ch)])
                pltpu.async_copy(table_hbm.at[idx_v.at[k_ + 4]],
                                 us[t], sgs[t])

        for k_ in range(c - 5, c):
            t = k_ % 4
            pltpu.make_async_copy(table_hbm.at[idx_v.at[k_]],
                                  us[t], sgs[t]).wait()
            pltpu.sync_copy(us[t], out_hbm.at[pl.ds(base + k_ * ch, ch)])
            if k_ + 4 < c:
                pltpu.async_copy(table_hbm.at[idx_v.at[k_ + 4]],
                                 us[t], sgs[t])

    return k(table, idx3)


def _sc_scatter(msg, dst3, zeros, n_acc, ep):
    nw, c, ch = dst3.shape
    epw = ep // _NW
    rpt = n_acc // _NS

    @functools.partial(
        pl.kernel,
        out_type=jax.ShapeDtypeStruct((_NC, n_acc, 16), jnp.float32),
        mesh=_worker_mesh(),
        compiler_params=pltpu.CompilerParams(use_tc_tiling_on_sc=False),
        scratch_types=[
            pltpu.VMEM((c, ch), jnp.int32),
            pltpu.VMEM((ch, 16), jnp.float32),
            pltpu.VMEM((ch, 16), jnp.float32),
            pltpu.VMEM_SHARED((n_acc, 16), jnp.float32),
            pltpu.SemaphoreType.DMA,
            pltpu.SemaphoreType.DMA,
        ],
    )
    def k(msg_hbm, dst_hbm, zero_hbm, out_hbm, idx_v, buf0, buf1, acc_sh,
          sem0, sem1):
        cid = lax.axis_index("c")
        sid = lax.axis_index("s")
        wid = sid * _NC + cid
        base = wid * epw
        pltpu.sync_copy(dst_hbm.at[wid], idx_v)
        pltpu.sync_copy(zero_hbm.at[pl.ds(sid * rpt, rpt)],
                        acc_sh.at[pl.ds(sid * rpt, rpt)])
        plsc.subcore_barrier()

        pltpu.async_copy(msg_hbm.at[pl.ds(base, ch)], buf0, sem0)

        @pl.loop(0, (c - 1) // 2)
        def _(j):
            k0 = 2 * j
            pltpu.async_copy(msg_hbm.at[pl.ds(base + (k0 + 1) * ch, ch)],
                             buf1, sem1)
            pltpu.make_async_copy(msg_hbm.at[pl.ds(base + k0 * ch, ch)],
                                  buf0, sem0).wait()
            pltpu.sync_copy(buf0, acc_sh.at[idx_v.at[k0]], add=True)
            pltpu.async_copy(msg_hbm.at[pl.ds(base + (k0 + 2) * ch, ch)],
                             buf0, sem0)
            pltpu.make_async_copy(msg_hbm.at[pl.ds(base + (k0 + 1) * ch, ch)],
                                  buf1, sem1).wait()
            pltpu.sync_copy(buf1, acc_sh.at[idx_v.at[k0 + 1]], add=True)

        pltpu.make_async_copy(msg_hbm.at[pl.ds(base + (c - 1) * ch, ch)],
                              buf0, sem0).wait()
        pltpu.sync_copy(buf0, acc_sh.at[idx_v.at[c - 1]], add=True)

        plsc.subcore_barrier()
        pltpu.sync_copy(acc_sh.at[pl.ds(sid * rpt, rpt)],
                        out_hbm.at[cid, pl.ds(sid * rpt, rpt)])

    return k(msg, dst3, zeros)


def _edge_msgs(hsrc_p, attr_t, w1, b1, w2, b2, in_dim, blk, ep):
    kdim = w2.shape[1]
    h_out = kdim // in_dim
    slots = 128 // in_dim
    grid = ep // blk
    pr = blk // slots

    ii = np.arange(kdim) // h_out
    exps = np.zeros((slots, 128, kdim), np.float32)
    for p in range(slots):
        exps[p, p * in_dim + ii, np.arange(kdim)] = 1.0
    exps = jnp.asarray(exps, jnp.bfloat16)
    oo = np.arange(kdim) % h_out
    select = jnp.asarray(oo[:, None] == np.arange(h_out)[None, :],
                         jnp.float32)

    def body(hsrc_ref, attr_ref, exp_ref, w1_ref, b1_ref, w2_ref, b2_ref,
             sel_ref, msg_ref):
        hb = hsrc_ref[...].astype(jnp.bfloat16)
        parts = [jnp.dot(hb, exp_ref[p], preferred_element_type=jnp.float32)
                 for p in range(slots)]
        hexp = jnp.concatenate(parts, axis=0)
        a = attr_ref[...].T.astype(jnp.bfloat16)
        z = jnp.dot(a, w1_ref[...],
                    preferred_element_type=jnp.float32) + b1_ref[...]
        u = jnp.maximum(z, 0.0).astype(jnp.bfloat16)
        wmat = jnp.dot(u, w2_ref[...],
                       preferred_element_type=jnp.float32) + b2_ref[...]
        prod = hexp * wmat
        msg = jnp.dot(prod, sel_ref[...], preferred_element_type=jnp.float32)
        s8 = blk // 8
        msg_ref[...] = jnp.concatenate(
            [msg[j * s8:(j + 1) * s8, :] for j in range(8)], axis=1)

    full = lambda shape: pl.BlockSpec(shape, lambda i: (0,) * len(shape))
    return pl.pallas_call(
        body,
        grid=(grid,),
        in_specs=[
            pl.BlockSpec((pr, 128), lambda i: (i, 0)),
            pl.BlockSpec((4, blk), lambda i: (0, i)),
            full(exps.shape),
            full(w1.shape),
            full((1, kdim)),
            full(w2.shape),
            full((1, kdim)),
            full(select.shape),
        ],
        out_specs=pl.BlockSpec((blk // 8, 128), lambda i: (i, 0)),
        out_shape=jax.ShapeDtypeStruct((ep * h_out // 128, 128), jnp.float32),
    )(hsrc_p, attr_t, exps, w1.astype(jnp.bfloat16), b1.reshape(1, -1),
      w2.astype(jnp.bfloat16), b2.reshape(1, -1), select)


def _bn_layer(aggp, h, root, bias, gamma, beta, n):
    h_dim = root.shape[1]

    def body(agg_ref, h_ref, root_ref, bias_ref, gamma_ref, beta_ref,
             hout_ref, colsum_ref):
        agg = agg_ref[0, :n, :] + agg_ref[1, :n, :]
        hpre = agg + jnp.dot(h_ref[...], root_ref[...],
                             preferred_element_type=jnp.float32) + bias_ref[...]
        mean = jnp.mean(hpre, axis=0, keepdims=True)
        var = jnp.mean((hpre - mean) ** 2, axis=0, keepdims=True)
        hn = (hpre - mean) * lax.rsqrt(var + _EPS) * gamma_ref[...] + beta_ref[...]
        hout = jnp.maximum(hn, 0.0)
        hout_ref[...] = hout
        colsum_ref[...] = jnp.sum(hout, axis=0, keepdims=True)

    return pl.pallas_call(
        body,
        out_shape=(jax.ShapeDtypeStruct((n, h_dim), jnp.float32),
                   jax.ShapeDtypeStruct((1, h_dim), jnp.float32)),
    )(aggp, h, root, bias.reshape(1, -1), gamma.reshape(1, -1),
      beta.reshape(1, -1))


def _head(colsum_cat, jump_w, jump_b, reg_w1, reg_b1, reg_w2, reg_b2, n):
    def body(cs_ref, jw_ref, jb_ref, w1_ref, b1_ref, w2_ref, b2_ref, out_ref):
        pooled = jnp.dot(cs_ref[...], jw_ref[...],
                         preferred_element_type=jnp.float32) + n * jb_ref[...]
        r = jnp.maximum(jnp.dot(pooled, w1_ref[...],
                                preferred_element_type=jnp.float32)
                        + b1_ref[...], 0.0)
        out_ref[...] = jnp.dot(r, w2_ref[...],
                               preferred_element_type=jnp.float32) + b2_ref[...]

    return pl.pallas_call(
        body,
        out_shape=jax.ShapeDtypeStruct((1, 1), jnp.float32),
    )(colsum_cat, jump_w, jump_b.reshape(1, -1), reg_w1, reg_b1.reshape(1, -1),
      reg_w2, reg_b2.reshape(1, -1))


def kernel(x, edge_index, edge_attr, params):
    n = x.shape[0]
    e = edge_attr.shape[0]
    src = edge_index[0]
    dst = edge_index[1]

    ep = -(-e // 4096) * 4096
    epw = ep // _NW
    c = epw // _CH
    n_acc = ((n + 1 + _NS - 1) // _NS) * _NS

    src_pad = jnp.pad(src, (0, ep - e))
    dst_pad = jnp.pad(dst, (0, ep - e), constant_values=n)
    zeros_acc = jnp.zeros((n_acc, 16), jnp.float32)
    attr_t = jnp.pad(edge_attr.T, ((0, 0), (0, ep - e)))

    h = x
    colsums = []
    for li, lp in enumerate(params["layers"]):
        in_dim = h.shape[1]
        slots = 128 // in_dim
        blk = 1024 if li == 0 else 4096
        pr = blk // slots
        src3 = src_pad.reshape(ep // blk, slots, pr) \
                      .swapaxes(1, 2).reshape(_NW, c, _CH)
        hsrc = _sc_gather(h, src3, ep)
        hsrc_p = hsrc.reshape(ep * in_dim // 128, 128)
        msg_p = _edge_msgs(hsrc_p, attr_t, lp["w1"], lp["b1"], lp["w2"],
                           lp["b2"], in_dim, blk, ep)
        dst3 = dst_pad.reshape(ep // blk, 8, blk // 8)                       .swapaxes(1, 2).reshape(_NW, c, _CH)
        aggp = _sc_scatter(msg_p.reshape(ep, 16), dst3, zeros_acc, n_acc, ep)
        h, cs = _bn_layer(aggp, h, lp["root"], lp["bias"], lp["gamma"],
                          lp["beta"], n)
        colsums.append(cs)

    cs_cat = jnp.concatenate(colsums, axis=1)
    return _head(cs_cat, params["jump_w"], params["jump_b"],
                 params["reg_w1"], params["reg_b1"],
                 params["reg_w2"], params["reg_b2"], float(n))
